# baseline (speedup 1.0000x reference)
import jax
import jax.numpy as jnp
from jax import lax
from jax.experimental import pallas as pl
from jax.experimental.pallas import tpu as pltpu
from jax.experimental.pallas import tpu_sc as plsc

B = 16384
F = 26
D = 4
NC, NS, L = 2, 16, 16
NW = NC * NS
EPW = B // NW
IPW = EPW * F
CH = 4
ECH = EPW // CH
ICH = ECH * F
GCH = ECH // L


def _fm_kernel(ids_hbm, bias_hbm, cross_hbm, lin_hbm,
               logits_hbm, adj_hbm, prob_hbm,
               idx_v,
               pv0, pv1,
               bias_v, logit_v, prob_v,
               sem0, sem1):
    wid = lax.axis_index("s") * NC + lax.axis_index("c")
    id_base = wid * IPW
    ex_base = wid * EPW

    bufs = [(pv0, sem0), (pv1, sem1)]

    def fire(c):
        pv, sems = bufs[c % 2]
        pltpu.sync_copy(ids_hbm.at[pl.ds(id_base + c * ICH, ICH)],
                        idx_v.at[c])
        idx = idx_v.at[c]
        nv = lin_hbm.shape[0]
        cps = []
        for d in range(D):
            cps.append(pltpu.async_copy(
                cross_hbm.at[pl.ds(d * nv, nv)].at[idx], pv.at[d], sems[d]))
        cps.append(pltpu.async_copy(
            lin_hbm.at[idx], pv.at[D], sems[D]))
        return cps

    pending = [fire(0), fire(1)]
    pltpu.sync_copy(bias_hbm, bias_v)

    iota = lax.iota(jnp.int32, L)
    row_base = iota * F
    d_c = [jnp.full((L,), d, jnp.int32) for d in range(D + 1)]
    bias_vec = bias_v[...]
    zero_f = jnp.zeros((L,), jnp.float32)

    for c in range(CH):
        pv, _ = bufs[c % 2]
        for cp in pending[c % 2]:
            cp.wait()

        def group_body(g, carry):
            r0 = row_base + g * (L * F)
            acc = [zero_f] * D
            accsq = [zero_f] * D
            lin = zero_f
            for f in range(F):
                r = r0 + f
                for d in range(D):
                    v = plsc.load_gather(pv, [d_c[d], r])
                    acc[d] = acc[d] + v
                    accsq[d] = accsq[d] + v * v
                lin = lin + plsc.load_gather(pv, [d_c[D], r])
            cross = zero_f
            for d in range(D):
                cross = cross + (acc[d] * acc[d] - accsq[d])
            logits = bias_vec + lin + 0.5 * cross
            prob = 1.0 / (1.0 + jnp.exp(-logits))
            logit_v[pl.ds(c * ECH + g * L, L)] = logits
            prob_v[pl.ds(c * ECH + g * L, L)] = prob
            return carry

        lax.fori_loop(0, GCH, group_body, 0)

        if c + 2 < CH:
            pending[c % 2] = fire(c + 2)

    pltpu.sync_copy(logit_v, logits_hbm.at[pl.ds(ex_base, EPW)])
    pltpu.sync_copy(logit_v, adj_hbm.at[pl.ds(ex_base, EPW)])
    pltpu.sync_copy(prob_v, prob_hbm.at[pl.ds(ex_base, EPW)])


@jax.jit
def kernel(feature_ids, linear_bias, linear_w, cross_emb_w):
    ids_flat = feature_ids.reshape(-1)
    bias16 = jnp.broadcast_to(linear_bias, (L,)).astype(jnp.float32)
    cross_t = cross_emb_w.T.reshape(-1)
    lin_t = linear_w.reshape(-1)

    run = pl.kernel(
        _fm_kernel,
        out_type=(
            jax.ShapeDtypeStruct((B,), jnp.float32),
            jax.ShapeDtypeStruct((B,), jnp.float32),
            jax.ShapeDtypeStruct((B,), jnp.float32),
        ),
        mesh=plsc.VectorSubcoreMesh(core_axis_name="c", subcore_axis_name="s"),
        compiler_params=pltpu.CompilerParams(
            needs_layout_passes=False, use_tc_tiling_on_sc=False),
        scratch_types=[
            pltpu.VMEM((CH, ICH), jnp.int32),
            pltpu.VMEM((D + 1, ICH), jnp.float32),
            pltpu.VMEM((D + 1, ICH), jnp.float32),
            pltpu.VMEM((L,), jnp.float32),
            pltpu.VMEM((EPW,), jnp.float32),
            pltpu.VMEM((EPW,), jnp.float32),
            [pltpu.SemaphoreType.DMA] * (D + 1),
            [pltpu.SemaphoreType.DMA] * (D + 1),
        ],
    )
    logits, adj, prob = run(ids_flat, bias16, cross_t, lin_t)
    return (logits[:, None], adj[:, None], prob[:, None])


# speedup vs baseline: 1.0010x; 1.0010x over previous
import jax
import jax.numpy as jnp
from jax import lax
from jax.experimental import pallas as pl
from jax.experimental.pallas import tpu as pltpu
from jax.experimental.pallas import tpu_sc as plsc

B = 16384
F = 26
D = 4
NC, NS, L = 2, 16, 16
NW = NC * NS
EPW = B // NW
IPW = EPW * F
CH = 4
ECH = EPW // CH
ICH = ECH * F
GCH = ECH // L


def _fm_kernel(ids_hbm, bias_hbm, cross_hbm, lin_hbm,
               logits_hbm, adj_hbm, prob_hbm,
               idx_v,
               pv0, pv1,
               bias_v, logit_v, prob_v,
               sem0, sem1):
    wid = lax.axis_index("s") * NC + lax.axis_index("c")
    id_base = wid * IPW
    ex_base = wid * EPW

    bufs = [(pv0, sem0), (pv1, sem1)]

    def fire(c):
        pv, sems = bufs[c % 2]
        pltpu.sync_copy(ids_hbm.at[pl.ds(id_base + c * ICH, ICH)],
                        idx_v.at[c])
        idx = idx_v.at[c]
        nv = lin_hbm.shape[0]
        cps = []
        for d in range(D):
            cps.append(pltpu.async_copy(
                cross_hbm.at[pl.ds(d * nv, nv)].at[idx], pv.at[d], sems[d]))
        cps.append(pltpu.async_copy(
            lin_hbm.at[idx], pv.at[D], sems[D]))
        return cps

    pending = [fire(0), fire(1)]
    pltpu.sync_copy(bias_hbm, bias_v)

    iota = lax.iota(jnp.int32, L)
    row_base = iota * F
    d_c = [jnp.full((L,), d, jnp.int32) for d in range(D + 1)]
    bias_vec = bias_v[...]
    zero_f = jnp.zeros((L,), jnp.float32)

    for c in range(CH):
        pv, _ = bufs[c % 2]
        for cp in pending[c % 2]:
            cp.wait()

        def group_body(g, carry):
            r0 = row_base + g * (L * F)
            acc = [zero_f] * D
            accsq = [zero_f] * D
            lin = zero_f
            for f in range(F):
                r = r0 + f
                for d in range(D):
                    v = plsc.load_gather(pv, [d_c[d], r])
                    acc[d] = acc[d] + v
                    accsq[d] = accsq[d] + v * v
                lin = lin + plsc.load_gather(pv, [d_c[D], r])
            cross = zero_f
            for d in range(D):
                cross = cross + (acc[d] * acc[d] - accsq[d])
            logits = bias_vec + lin + 0.5 * cross
            prob = 1.0 / (1.0 + jnp.exp(-logits))
            logit_v[pl.ds(c * ECH + g * L, L)] = logits
            prob_v[pl.ds(c * ECH + g * L, L)] = prob
            return carry

        lax.fori_loop(0, GCH, group_body, 0)

        if c + 2 < CH:
            pending[c % 2] = fire(c + 2)

    pltpu.sync_copy(logit_v, logits_hbm.at[pl.ds(ex_base, EPW)])
    pltpu.sync_copy(logit_v, adj_hbm.at[pl.ds(ex_base, EPW)])
    pltpu.sync_copy(prob_v, prob_hbm.at[pl.ds(ex_base, EPW)])


@jax.jit
def kernel(feature_ids, linear_bias, linear_w, cross_emb_w):
    ids_flat = feature_ids.reshape(-1)
    bias16 = jnp.broadcast_to(linear_bias, (L,)).astype(jnp.float32)
    cross_t = cross_emb_w.T.reshape(-1)
    lin_t = linear_w[:, 0]

    run = pl.kernel(
        _fm_kernel,
        out_type=(
            jax.ShapeDtypeStruct((B,), jnp.float32),
            jax.ShapeDtypeStruct((B,), jnp.float32),
            jax.ShapeDtypeStruct((B,), jnp.float32),
        ),
        mesh=plsc.VectorSubcoreMesh(core_axis_name="c", subcore_axis_name="s"),
        compiler_params=pltpu.CompilerParams(
            needs_layout_passes=False, use_tc_tiling_on_sc=False),
        scratch_types=[
            pltpu.VMEM((CH, ICH), jnp.int32),
            pltpu.VMEM((D + 1, ICH), jnp.float32),
            pltpu.VMEM((D + 1, ICH), jnp.float32),
            pltpu.VMEM((L,), jnp.float32),
            pltpu.VMEM((EPW,), jnp.float32),
            pltpu.VMEM((EPW,), jnp.float32),
            [pltpu.SemaphoreType.DMA] * (D + 1),
            [pltpu.SemaphoreType.DMA] * (D + 1),
        ],
    )
    logits, adj, prob = run(ids_flat, bias16, cross_t, lin_t)
    return (logits[:, None], adj[:, None], prob[:, None])
